# Initial kernel scaffold; baseline (speedup 1.0000x reference)
import jax
import jax.numpy as jnp
from jax.experimental import pallas as pl


def kernel(ast_nodes_embeddings, edge_index, W_iou, U_iou, b_iou, W_f, U_f, b_f):
    raise NotImplementedError("write your pallas kernel here")


# speedup vs baseline: 3.6600x; 3.6600x over previous
import functools

import jax
import jax.numpy as jnp
from jax import lax
from jax.experimental import pallas as pl
from jax.experimental.pallas import tpu as pltpu
from jax.experimental.pallas import tpu_sc as plsc

N_NODES = 10000
D = 128
N_EDGES = 320000
NUM_STEPS = 8

NP = 10240
EP = 320640
K = 120
CHUNKS = EP // (16 * K)
EPT = CHUNKS * K
RPT = NP // 16
H = 64


def _tc_init_body(x_ref, wiou_ref, biou_ref, wf_ref, bf_ref, uf_ref,
                  xiou_ref, xf2_ref, hhuf_ref, c2_ref):
    x = x_ref[...]
    xiou = jnp.dot(x, wiou_ref[...], preferred_element_type=jnp.float32)
    xiou = xiou + biou_ref[...]
    xiou_ref[...] = xiou
    xf = jnp.dot(x, wf_ref[...], preferred_element_type=jnp.float32)
    xf = xf + bf_ref[...]
    i = jax.nn.sigmoid(xiou[:, 0:D])
    o = jax.nn.sigmoid(xiou[:, D:2 * D])
    u = jnp.tanh(xiou[:, 2 * D:3 * D])
    c1 = i * u
    h1 = o * jnp.tanh(c1)
    huf = jnp.dot(h1, uf_ref[...], preferred_element_type=jnp.float32)
    xf2_ref[0] = xf
    xf2_ref[1] = jnp.concatenate([xf[:, H:], xf[:, :H]], axis=1)
    hhuf_ref[0] = jnp.concatenate([h1[:, :H], huf[:, :H]], axis=1)
    hhuf_ref[1] = jnp.concatenate([h1[:, H:], huf[:, H:]], axis=1)
    c2_ref[0] = c1
    c2_ref[1] = jnp.concatenate([c1[:, H:], c1[:, :H]], axis=1)


_BLK = 1024
_GRID = NP // _BLK


def _tc_init(x_pad, W_iou, b_iou2, W_f, b_f2, U_f):
    half_spec = pl.BlockSpec((2, _BLK, D), lambda i: (0, i, 0))
    return pl.pallas_call(
        _tc_init_body,
        grid=(_GRID,),
        in_specs=[
            pl.BlockSpec((_BLK, D), lambda i: (i, 0)),
            pl.BlockSpec((D, 3 * D), lambda i: (0, 0)),
            pl.BlockSpec((1, 3 * D), lambda i: (0, 0)),
            pl.BlockSpec((D, D), lambda i: (0, 0)),
            pl.BlockSpec((1, D), lambda i: (0, 0)),
            pl.BlockSpec((D, D), lambda i: (0, 0)),
        ],
        out_specs=[
            pl.BlockSpec((_BLK, 3 * D), lambda i: (i, 0)),
            half_spec, half_spec, half_spec,
        ],
        out_shape=[
            jax.ShapeDtypeStruct((NP, 3 * D), jnp.float32),
            jax.ShapeDtypeStruct((2, NP, D), jnp.float32),
            jax.ShapeDtypeStruct((2, NP, D), jnp.float32),
            jax.ShapeDtypeStruct((2, NP, D), jnp.float32),
        ],
    )(x_pad, W_iou, b_iou2, W_f, b_f2, U_f)


def _tc_step_body(xiou_ref, acc_ref, uiou_ref, uf_ref,
                  hhuf_ref, c2_ref):
    a0 = acc_ref[0]
    a1 = acc_ref[1]
    hs = jnp.concatenate([a0[:, :H], a1[:, :H]], axis=1)
    mix = jnp.concatenate([a0[:, H:], a1[:, H:]], axis=1)
    fc = mix - jnp.dot(hs, uf_ref[...], preferred_element_type=jnp.float32)
    iou = xiou_ref[...] + jnp.dot(hs, uiou_ref[...],
                                  preferred_element_type=jnp.float32)
    i = jax.nn.sigmoid(iou[:, 0:D])
    o = jax.nn.sigmoid(iou[:, D:2 * D])
    u = jnp.tanh(iou[:, 2 * D:3 * D])
    c_new = i * u + fc
    h_new = o * jnp.tanh(c_new)
    huf = jnp.dot(h_new, uf_ref[...], preferred_element_type=jnp.float32)
    hhuf_ref[0] = jnp.concatenate([h_new[:, :H], huf[:, :H]], axis=1)
    hhuf_ref[1] = jnp.concatenate([h_new[:, H:], huf[:, H:]], axis=1)
    c2_ref[0] = c_new
    c2_ref[1] = jnp.concatenate([c_new[:, H:], c_new[:, :H]], axis=1)


def _tc_step(x_iou, acc3, U_iou, U_f):
    half_spec = pl.BlockSpec((2, _BLK, D), lambda i: (0, i, 0))
    return pl.pallas_call(
        _tc_step_body,
        grid=(_GRID,),
        in_specs=[
            pl.BlockSpec((_BLK, 3 * D), lambda i: (i, 0)),
            half_spec,
            pl.BlockSpec((D, 3 * D), lambda i: (0, 0)),
            pl.BlockSpec((D, D), lambda i: (0, 0)),
        ],
        out_specs=[half_spec, half_spec],
        out_shape=[
            jax.ShapeDtypeStruct((2, NP, D), jnp.float32),
            jax.ShapeDtypeStruct((2, NP, D), jnp.float32),
        ],
    )(x_iou, acc3, U_iou, U_f)


def _sc_edge_body(src2_hbm, dst2_hbm, dstr_hbm, hhuf_hbm, c2_hbm, xf2_hbm,
                  z_hbm, out_hbm,
                  idx_s, idx_d2, idx_dr, hhuf_rows, c_rows, xf_rows,
                  acc, sem):
    core = lax.axis_index("c")
    sub = lax.axis_index("s")
    row0 = sub * RPT
    pltpu.sync_copy(z_hbm, acc.at[pl.ds(row0, RPT)])
    plsc.subcore_barrier()

    e0 = sub * EPT
    eoff = core * EP + e0

    def chunk(i, carry):
        off = i * K
        pltpu.sync_copy(src2_hbm.at[pl.ds(eoff + off, K)], idx_s)
        pltpu.sync_copy(dst2_hbm.at[pl.ds(eoff + off, K)], idx_d2)
        pltpu.sync_copy(dstr_hbm.at[pl.ds(e0 + off, K)], idx_dr)
        cp1 = pltpu.async_copy(hhuf_hbm.at[idx_s], hhuf_rows, sem)
        cp2 = pltpu.async_copy(c2_hbm.at[idx_s], c_rows, sem)
        cp3 = pltpu.async_copy(xf2_hbm.at[idx_d2], xf_rows, sem)
        cp1.wait()
        cp2.wait()
        cp3.wait()

        def erow(e, c2):
            for j in range(H // 16):
                slo = pl.ds(16 * j, 16)
                shi = pl.ds(H + 16 * j, 16)
                huf = hhuf_rows[e, shi]
                t = xf_rows[e, slo] + huf
                hhuf_rows[e, shi] = huf + c_rows[e, slo] / (1.0 + jnp.exp(-t))
            return c2

        lax.fori_loop(0, K, erow, 0)
        pltpu.sync_copy(hhuf_rows, acc.at[idx_dr], add=True)
        return carry

    lax.fori_loop(0, CHUNKS, chunk, 0)
    plsc.subcore_barrier()
    pltpu.sync_copy(acc.at[pl.ds(row0, RPT)],
                    out_hbm.at[pl.ds(core * NP + row0, RPT)])


@functools.cache
def _make_sc_edge():
    return pl.kernel(
        _sc_edge_body,
        mesh=plsc.VectorSubcoreMesh(core_axis_name="c", subcore_axis_name="s"),
        out_type=jax.ShapeDtypeStruct((2 * NP, D), jnp.float32),
        scratch_types=[
            pltpu.VMEM((K,), jnp.int32),
            pltpu.VMEM((K,), jnp.int32),
            pltpu.VMEM((K,), jnp.int32),
            pltpu.VMEM((K, D), jnp.float32),
            pltpu.VMEM((K, D), jnp.float32),
            pltpu.VMEM((K, D), jnp.float32),
            pltpu.VMEM_SHARED((NP, D), jnp.float32),
            pltpu.SemaphoreType.DMA,
        ],
    )


def _sc_edge(*args):
    return _make_sc_edge()(*args)


def kernel(ast_nodes_embeddings, edge_index, W_iou, U_iou, b_iou, W_f,
           U_f, b_f):
    x = ast_nodes_embeddings
    src = edge_index[0].astype(jnp.int32)
    dst = edge_index[1].astype(jnp.int32)

    padlen = EP - N_EDGES
    pad_rows = N_NODES + (jnp.arange(padlen, dtype=jnp.int32) % 128)
    src_p = jnp.concatenate([src, pad_rows])
    dst_p = jnp.concatenate([dst, pad_rows])
    src2 = jnp.concatenate([src_p, src_p + NP])
    dst2 = jnp.concatenate([dst_p, dst_p + NP])
    zrows = jnp.zeros((RPT, D), jnp.float32)

    x_pad = jnp.pad(x, ((0, NP - N_NODES), (0, 0)))
    b_iou2 = b_iou.reshape(1, 3 * D)
    b_f2 = b_f.reshape(1, D)

    x_iou, xf3, hhuf3, c3 = _tc_init(x_pad, W_iou, b_iou2, W_f, b_f2, U_f)
    xf_t = xf3.reshape(2 * NP, D)

    for _ in range(NUM_STEPS - 1):
        out_acc = _sc_edge(src2, dst2, dst_p, hhuf3.reshape(2 * NP, D),
                           c3.reshape(2 * NP, D), xf_t, zrows)
        hhuf3, c3 = _tc_step(x_iou, out_acc.reshape(2, NP, D), U_iou, U_f)

    return jnp.concatenate([hhuf3[0, :N_NODES, :H], hhuf3[1, :N_NODES, :H]],
                           axis=1)


# speedup vs baseline: 3.7276x; 1.0185x over previous
import functools

import jax
import jax.numpy as jnp
from jax import lax
from jax.experimental import pallas as pl
from jax.experimental.pallas import tpu as pltpu
from jax.experimental.pallas import tpu_sc as plsc

N_NODES = 10000
D = 128
N_EDGES = 320000
NUM_STEPS = 8

NP = 10240
EP = 320640
K = 120
CHUNKS = EP // (16 * K)
EPT = CHUNKS * K
RPT = NP // 16
H = 64


def _tc_init_body(x_ref, wiou_ref, biou_ref, wf_ref, bf_ref, uf_ref,
                  xiou_ref, xf2_ref, hhuf_ref, c2_ref):
    x = x_ref[...]
    xiou = jnp.dot(x, wiou_ref[...], preferred_element_type=jnp.float32)
    xiou = xiou + biou_ref[...]
    xiou_ref[...] = xiou
    xf = jnp.dot(x, wf_ref[...], preferred_element_type=jnp.float32)
    xf = xf + bf_ref[...]
    i = jax.nn.sigmoid(xiou[:, 0:D])
    o = jax.nn.sigmoid(xiou[:, D:2 * D])
    u = jnp.tanh(xiou[:, 2 * D:3 * D])
    c1 = i * u
    h1 = o * jnp.tanh(c1)
    huf = jnp.dot(h1, -uf_ref[...], preferred_element_type=jnp.float32)
    xf2_ref[0] = -xf
    xf2_ref[1] = jnp.concatenate([-xf[:, H:], -xf[:, :H]], axis=1)
    hhuf_ref[0] = jnp.concatenate([h1[:, :H], huf[:, :H]], axis=1)
    hhuf_ref[1] = jnp.concatenate([h1[:, H:], huf[:, H:]], axis=1)
    c2_ref[0] = c1
    c2_ref[1] = jnp.concatenate([c1[:, H:], c1[:, :H]], axis=1)


_BLK = 1024
_GRID = NP // _BLK


def _tc_init(x_pad, W_iou, b_iou2, W_f, b_f2, U_f):
    half_spec = pl.BlockSpec((2, _BLK, D), lambda i: (0, i, 0))
    return pl.pallas_call(
        _tc_init_body,
        grid=(_GRID,),
        in_specs=[
            pl.BlockSpec((_BLK, D), lambda i: (i, 0)),
            pl.BlockSpec((D, 3 * D), lambda i: (0, 0)),
            pl.BlockSpec((1, 3 * D), lambda i: (0, 0)),
            pl.BlockSpec((D, D), lambda i: (0, 0)),
            pl.BlockSpec((1, D), lambda i: (0, 0)),
            pl.BlockSpec((D, D), lambda i: (0, 0)),
        ],
        out_specs=[
            pl.BlockSpec((_BLK, 3 * D), lambda i: (i, 0)),
            half_spec, half_spec, half_spec,
        ],
        out_shape=[
            jax.ShapeDtypeStruct((NP, 3 * D), jnp.float32),
            jax.ShapeDtypeStruct((2, NP, D), jnp.float32),
            jax.ShapeDtypeStruct((2, NP, D), jnp.float32),
            jax.ShapeDtypeStruct((2, NP, D), jnp.float32),
        ],
    )(x_pad, W_iou, b_iou2, W_f, b_f2, U_f)


def _tc_step_body(xiou_ref, acc_ref, uiou_ref, uf_ref,
                  hhuf_ref, c2_ref):
    a0 = acc_ref[0]
    a1 = acc_ref[1]
    hs = jnp.concatenate([a0[:, :H], a1[:, :H]], axis=1)
    mix = jnp.concatenate([a0[:, H:], a1[:, H:]], axis=1)
    fc = mix + jnp.dot(hs, uf_ref[...], preferred_element_type=jnp.float32)
    iou = xiou_ref[...] + jnp.dot(hs, uiou_ref[...],
                                  preferred_element_type=jnp.float32)
    i = jax.nn.sigmoid(iou[:, 0:D])
    o = jax.nn.sigmoid(iou[:, D:2 * D])
    u = jnp.tanh(iou[:, 2 * D:3 * D])
    c_new = i * u + fc
    h_new = o * jnp.tanh(c_new)
    huf = jnp.dot(h_new, -uf_ref[...], preferred_element_type=jnp.float32)
    hhuf_ref[0] = jnp.concatenate([h_new[:, :H], huf[:, :H]], axis=1)
    hhuf_ref[1] = jnp.concatenate([h_new[:, H:], huf[:, H:]], axis=1)
    c2_ref[0] = c_new
    c2_ref[1] = jnp.concatenate([c_new[:, H:], c_new[:, :H]], axis=1)


def _tc_step(x_iou, acc3, U_iou, U_f):
    half_spec = pl.BlockSpec((2, _BLK, D), lambda i: (0, i, 0))
    return pl.pallas_call(
        _tc_step_body,
        grid=(_GRID,),
        in_specs=[
            pl.BlockSpec((_BLK, 3 * D), lambda i: (i, 0)),
            half_spec,
            pl.BlockSpec((D, 3 * D), lambda i: (0, 0)),
            pl.BlockSpec((D, D), lambda i: (0, 0)),
        ],
        out_specs=[half_spec, half_spec],
        out_shape=[
            jax.ShapeDtypeStruct((2, NP, D), jnp.float32),
            jax.ShapeDtypeStruct((2, NP, D), jnp.float32),
        ],
    )(x_iou, acc3, U_iou, U_f)


def _sc_edge_body(src2_hbm, dst2_hbm, dstr_hbm, hhuf_hbm, c2_hbm, xf2_hbm,
                  z_hbm, out_hbm,
                  idx_s, idx_d2, idx_dr, hhuf_rows, c_rows, xf_rows,
                  acc, sem):
    core = lax.axis_index("c")
    sub = lax.axis_index("s")
    row0 = sub * RPT
    pltpu.sync_copy(z_hbm, acc.at[pl.ds(row0, RPT)])
    plsc.subcore_barrier()

    e0 = sub * EPT
    eoff = core * EP + e0

    def chunk(i, carry):
        off = i * K
        pltpu.sync_copy(src2_hbm.at[pl.ds(eoff + off, K)], idx_s)
        pltpu.sync_copy(dst2_hbm.at[pl.ds(eoff + off, K)], idx_d2)
        pltpu.sync_copy(dstr_hbm.at[pl.ds(e0 + off, K)], idx_dr)
        cp1 = pltpu.async_copy(hhuf_hbm.at[idx_s], hhuf_rows, sem)
        cp2 = pltpu.async_copy(c2_hbm.at[idx_s], c_rows, sem)
        cp3 = pltpu.async_copy(xf2_hbm.at[idx_d2], xf_rows, sem)
        cp1.wait()
        cp2.wait()
        cp3.wait()

        def erow(e, c2):
            for j in range(H // 16):
                slo = pl.ds(16 * j, 16)
                shi = pl.ds(H + 16 * j, 16)
                nhuf = hhuf_rows[e, shi]
                t = xf_rows[e, slo] + nhuf
                hhuf_rows[e, shi] = (
                    nhuf + c_rows[e, slo] / (1.0 + jnp.exp(t)))
            return c2

        lax.fori_loop(0, K, erow, 0)
        pltpu.sync_copy(hhuf_rows, acc.at[idx_dr], add=True)
        return carry

    lax.fori_loop(0, CHUNKS, chunk, 0)
    plsc.subcore_barrier()
    pltpu.sync_copy(acc.at[pl.ds(row0, RPT)],
                    out_hbm.at[pl.ds(core * NP + row0, RPT)])


@functools.cache
def _make_sc_edge():
    return pl.kernel(
        _sc_edge_body,
        mesh=plsc.VectorSubcoreMesh(core_axis_name="c", subcore_axis_name="s"),
        out_type=jax.ShapeDtypeStruct((2 * NP, D), jnp.float32),
        scratch_types=[
            pltpu.VMEM((K,), jnp.int32),
            pltpu.VMEM((K,), jnp.int32),
            pltpu.VMEM((K,), jnp.int32),
            pltpu.VMEM((K, D), jnp.float32),
            pltpu.VMEM((K, D), jnp.float32),
            pltpu.VMEM((K, D), jnp.float32),
            pltpu.VMEM_SHARED((NP, D), jnp.float32),
            pltpu.SemaphoreType.DMA,
        ],
    )


def _sc_edge(*args):
    return _make_sc_edge()(*args)


def kernel(ast_nodes_embeddings, edge_index, W_iou, U_iou, b_iou, W_f,
           U_f, b_f):
    x = ast_nodes_embeddings
    src = edge_index[0].astype(jnp.int32)
    dst = edge_index[1].astype(jnp.int32)

    padlen = EP - N_EDGES
    pad_rows = N_NODES + (jnp.arange(padlen, dtype=jnp.int32) % 128)
    src_p = jnp.concatenate([src, pad_rows])
    dst_p = jnp.concatenate([dst, pad_rows])
    src2 = jnp.concatenate([src_p, src_p + NP])
    dst2 = jnp.concatenate([dst_p, dst_p + NP])
    zrows = jnp.zeros((RPT, D), jnp.float32)

    x_pad = jnp.pad(x, ((0, NP - N_NODES), (0, 0)))
    b_iou2 = b_iou.reshape(1, 3 * D)
    b_f2 = b_f.reshape(1, D)

    x_iou, xf3, hhuf3, c3 = _tc_init(x_pad, W_iou, b_iou2, W_f, b_f2, U_f)
    xf_t = xf3.reshape(2 * NP, D)

    for _ in range(NUM_STEPS - 1):
        out_acc = _sc_edge(src2, dst2, dst_p, hhuf3.reshape(2 * NP, D),
                           c3.reshape(2 * NP, D), xf_t, zrows)
        hhuf3, c3 = _tc_step(x_iou, out_acc.reshape(2, NP, D), U_iou, U_f)

    return jnp.concatenate([hhuf3[0, :N_NODES, :H], hhuf3[1, :N_NODES, :H]],
                           axis=1)
